# baseline (speedup 1.0000x reference)
import functools

import jax
import jax.numpy as jnp
from jax import lax
from jax.experimental import pallas as pl
from jax.experimental.pallas import tpu as pltpu
from jax.experimental.pallas import tpu_sc as plsc

N0, N1, KNN = 50000, 12500, 16
N0P = 51200
N1P = 12800
N0BP = 51200
_S = float((1.0 + 1e-5) ** -0.5)
_NW = 32
D = 128


def _gelu(v):
    return 0.5 * v * (1.0 + lax.erf(v * 0.7071067811865476))


def _sc_gather_fn(V, B, C, dt):
    b_per_w = B // _NW
    nch = b_per_w // C
    mesh = plsc.VectorSubcoreMesh(core_axis_name="c", subcore_axis_name="s")

    @functools.partial(
        pl.kernel,
        out_type=jax.ShapeDtypeStruct((B, D), dt),
        mesh=mesh,
        scratch_types=[
            pltpu.VMEM((C,), jnp.int32),
            pltpu.VMEM((C,), jnp.int32),
            pltpu.VMEM((C, D), dt),
            pltpu.VMEM((C, D), dt),
            pltpu.SemaphoreType.DMA,
            pltpu.SemaphoreType.DMA,
            pltpu.SemaphoreType.DMA,
            pltpu.SemaphoreType.DMA,
        ],
    )
    def gather_k(table_hbm, idx_hbm, out_hbm, i0, i1, r0, r1, g0, g1, w0, w1):
        wid = lax.axis_index("s") * 2 + lax.axis_index("c")
        wbase = wid * b_per_w
        ibufs, rbufs, gsems, wsems = (i0, i1), (r0, r1), (g0, g1), (w0, w1)

        def start_gather(b, c):
            base = wbase + c * C
            pltpu.sync_copy(idx_hbm.at[pl.ds(base, C)], ibufs[b])
            pltpu.async_copy(table_hbm.at[ibufs[b]], rbufs[b], gsems[b])

        if nch < 2:
            pltpu.sync_copy(idx_hbm.at[pl.ds(wbase, C)], i0)
            pltpu.async_copy(table_hbm.at[i0], r0, g0).wait()
            pltpu.sync_copy(r0, out_hbm.at[pl.ds(wbase, C)])
            return

        start_gather(0, 0)
        start_gather(1, 1)

        def pair_body(j, carry):
            for b in range(2):
                c = 2 * j + b
                base = wbase + c * C
                pltpu.make_async_copy(table_hbm.at[pl.ds(0, C)], rbufs[b],
                                      gsems[b]).wait()
                pltpu.async_copy(rbufs[b], out_hbm.at[pl.ds(base, C)], wsems[b])
                pltpu.make_async_copy(rbufs[b], out_hbm.at[pl.ds(base, C)],
                                      wsems[b]).wait()

                @pl.when(c + 2 < nch)
                def _next():
                    start_gather(b, c + 2)

            return carry

        lax.fori_loop(0, nch // 2, pair_body, 0)

    return gather_k


def _gather_rows(table, idx, chunk):
    V, d = table.shape
    B = idx.shape[0]
    assert d == D and B % (_NW * chunk) == 0 and chunk % 8 == 0, (V, d, B, chunk)
    return _sc_gather_fn(V, B, chunk, table.dtype)(table, idx)


def _sc_gather4_fn(V, B, C):
    b_per_w = B // _NW
    assert b_per_w == C
    mesh = plsc.VectorSubcoreMesh(core_axis_name="c", subcore_axis_name="s")
    out4 = tuple(jax.ShapeDtypeStruct((B, D), jnp.float32) for _ in range(4))

    @functools.partial(
        pl.kernel,
        out_type=out4,
        mesh=mesh,
        scratch_types=[
            pltpu.VMEM((C,), jnp.int32),
            pltpu.VMEM((C, D), jnp.float32),
            pltpu.SemaphoreType.DMA,
        ],
    )
    def gather4_k(t0_h, t1_h, t2_h, t3_h, idx_hbm, o0, o1, o2, o3,
                  idx_v, rows_v, sem):
        wid = lax.axis_index("s") * 2 + lax.axis_index("c")
        wbase = wid * b_per_w
        pltpu.sync_copy(idx_hbm.at[pl.ds(wbase, C)], idx_v)
        for t_h, o_h in ((t0_h, o0), (t1_h, o1), (t2_h, o2), (t3_h, o3)):
            pltpu.async_copy(t_h.at[idx_v], rows_v, sem).wait()
            pltpu.sync_copy(rows_v, o_h.at[pl.ds(wbase, C)])

    return gather4_k


def _gather_rows4(tables, idx):
    B = idx.shape[0]
    return _sc_gather4_fn(tables[0].shape[0], B, B // _NW)(*tables, idx)


def _row_bs(R, d):
    return pl.BlockSpec((R, d), lambda i: (i, 0))


def _full_bs(shape):
    nd = len(shape)
    return pl.BlockSpec(shape, lambda i: (0,) * nd)


def _edge_bs(R):
    return pl.BlockSpec((KNN, R, D), lambda i: (0, i, 0))


def _dot(a, b):
    return jnp.dot(a, b, preferred_element_type=jnp.float32)


def _tc_matmul(x, w, R):
    N, Di = x.shape
    Do = w.shape[1]

    def body(x_ref, w_ref, o_ref):
        o_ref[...] = _dot(x_ref[...], w_ref[...])

    return pl.pallas_call(
        body,
        grid=(N // R,),
        in_specs=[_row_bs(R, Di), _full_bs(w.shape)],
        out_specs=_row_bs(R, Do),
        out_shape=jax.ShapeDtypeStruct((N, Do), jnp.float32),
    )(x, w)


def _tc_mm2(x, wa, wb, R):
    N, Di = x.shape

    def body(x_ref, wa_ref, wb_ref, oa_ref, ob_ref):
        v = x_ref[...]
        oa_ref[...] = _dot(v, wa_ref[...])
        ob_ref[...] = _dot(v, wb_ref[...])

    return pl.pallas_call(
        body,
        grid=(N // R,),
        in_specs=[_row_bs(R, Di), _full_bs(wa.shape), _full_bs(wb.shape)],
        out_specs=(_row_bs(R, wa.shape[1]), _row_bs(R, wb.shape[1])),
        out_shape=(jax.ShapeDtypeStruct((N, wa.shape[1]), jnp.float32),
                   jax.ShapeDtypeStruct((N, wb.shape[1]), jnp.float32)),
    )(x, wa, wb)


def _tc_mlp_mm(x, w1, b1, w2s, nxtw, R):
    N, d = x.shape

    def body(x_ref, w1_ref, b1_ref, w2_ref, nw_ref, o_ref, y_ref):
        v = x_ref[...]
        t = v + _dot(_gelu(_dot(v, w1_ref[...]) + b1_ref[...]), w2_ref[...])
        o_ref[...] = t
        y_ref[...] = _dot(t, nw_ref[...])

    return pl.pallas_call(
        body,
        grid=(N // R,),
        in_specs=[_row_bs(R, d), _full_bs(w1.shape), _full_bs((1,) + b1.shape),
                  _full_bs(w2s.shape), _full_bs(nxtw.shape)],
        out_specs=(_row_bs(R, d), _row_bs(R, d)),
        out_shape=(jax.ShapeDtypeStruct((N, d), jnp.float32),
                   jax.ShapeDtypeStruct((N, d), jnp.float32)),
    )(x, w1, b1[None, :], w2s, nxtw)


def _tc_lfp_step(g3, y, f, svec, mlp, nxtw, R):
    N = y.shape[0]
    specs = [_edge_bs(R), _row_bs(R, D), _row_bs(R, D), _full_bs((1, D))]
    args = [g3, y, f, svec[None, :]]
    if mlp is not None:
        w1, b1, w2s = mlp
        specs += [_full_bs(w1.shape), _full_bs((1,) + b1.shape), _full_bs(w2s.shape)]
        args += [w1, b1[None, :], w2s]
    if nxtw is not None:
        specs += [_full_bs(nxtw.shape)]
        args += [nxtw]

    def body(*refs):
        g_ref, y_ref, f_ref, s_ref = refs[:4]
        rest = list(refs[4:-2] if nxtw is not None else refs[4:-1])
        acc = g_ref[0]
        for k in range(1, KNN):
            acc = jnp.maximum(acc, g_ref[k])
        t = f_ref[...] + (acc - y_ref[...]) * s_ref[...]
        if mlp is not None:
            w1_ref, b1_ref, w2_ref = rest[:3]
            t = t + _dot(_gelu(_dot(t, w1_ref[...]) + b1_ref[...]), w2_ref[...])
        if nxtw is not None:
            refs[-2][...] = t
            refs[-1][...] = _dot(t, refs[4 + (3 if mlp is not None else 0)][...])
        else:
            refs[-1][...] = t

    if nxtw is not None:
        out_specs = (_row_bs(R, D), _row_bs(R, D))
        out_shape = (jax.ShapeDtypeStruct((N, D), jnp.float32),
                     jax.ShapeDtypeStruct((N, D), jnp.float32))
    else:
        out_specs = _row_bs(R, D)
        out_shape = jax.ShapeDtypeStruct((N, D), jnp.float32)

    return pl.pallas_call(
        body,
        grid=(N // R,),
        in_specs=specs,
        out_specs=out_specs,
        out_shape=out_shape,
    )(*args)


def _tc_embed3(x, w1, w2, w3, add, R):
    N, Di = x.shape
    Do = w3.shape[1]

    def body(x_ref, w1_ref, w2_ref, w3_ref, a_ref, o_ref):
        h = _gelu(_dot(x_ref[...], w1_ref[...]))
        h = _gelu(_dot(h, w2_ref[...]))
        o_ref[...] = _dot(h, w3_ref[...]) + a_ref[...]

    return pl.pallas_call(
        body,
        grid=(N // R,),
        in_specs=[_row_bs(R, Di), _full_bs(w1.shape), _full_bs(w2.shape),
                  _full_bs(w3.shape), _row_bs(R, Do)],
        out_specs=_row_bs(R, Do),
        out_shape=jax.ShapeDtypeStruct((N, Do), jnp.float32),
    )(x, w1, w2, w3, add)


def _tc_lfp_max(g3, y, res, svec, R):
    N = y.shape[0]

    def body(g_ref, y_ref, r_ref, s_ref, o_ref):
        acc = g_ref[0]
        for k in range(1, KNN):
            acc = jnp.maximum(acc, g_ref[k])
        o_ref[...] = r_ref[...] + (acc.astype(jnp.float32) - y_ref[...]) * s_ref[...]

    return pl.pallas_call(
        body,
        grid=(N // R,),
        in_specs=[_edge_bs(R), _row_bs(R, D), _row_bs(R, D), _full_bs((1, D))],
        out_specs=_row_bs(R, D),
        out_shape=jax.ShapeDtypeStruct((N, D), jnp.float32),
    )(g3, y, res, svec[None, :])


def _tc_edge0(tg, center16, w1, w2, w3, svec, N, R):
    DZ = w3.shape[1]

    def body(tg_ref, c_ref, w1_ref, w2_ref, w3_ref, s_ref, o_ref):
        c = c_ref[...]
        acc = None
        for k in range(KNN):
            e = tg_ref[k, :, 0:16] - c
            h = _gelu(_dot(e, w1_ref[...]))
            h = _gelu(_dot(h, w2_ref[...]))
            h = _dot(h, w3_ref[...]) + tg_ref[k, :, 16:16 + DZ]
            acc = h if acc is None else jnp.maximum(acc, h)
        o_ref[...] = jnp.concatenate(
            [acc * s_ref[...], jnp.zeros((acc.shape[0], D - DZ), jnp.float32)], axis=1)

    return pl.pallas_call(
        body,
        grid=(N // R,),
        in_specs=[_edge_bs(R), _row_bs(R, 16), _full_bs(w1.shape), _full_bs(w2.shape),
                  _full_bs(w3.shape), _full_bs((1, DZ))],
        out_specs=_row_bs(R, D),
        out_shape=jax.ShapeDtypeStruct((N, D), jnp.float32),
    )(tg, center16, w1, w2, w3, svec[None, :])


def _tc_edge1(tg, center16, w1, w2, w3, wproj, res, N, R):
    DZ = w3.shape[1]

    def body(tg_ref, c_ref, w1_ref, w2_ref, w3_ref, wp_ref, r_ref, o_ref):
        c = c_ref[...]
        acc = None
        for k in range(KNN):
            e = tg_ref[k, :, 0:16] - c
            h = _gelu(_dot(e, w1_ref[...]))
            h = _gelu(_dot(h, w2_ref[...]))
            h = _dot(h, w3_ref[...]) + tg_ref[k, :, 16:16 + DZ]
            acc = h if acc is None else jnp.maximum(acc, h)
        o_ref[...] = r_ref[...] + _dot(acc, wp_ref[...])

    return pl.pallas_call(
        body,
        grid=(N // R,),
        in_specs=[_edge_bs(R), _row_bs(R, 16), _full_bs(w1.shape), _full_bs(w2.shape),
                  _full_bs(w3.shape), _full_bs(wproj.shape), _row_bs(R, D)],
        out_specs=_row_bs(R, D),
        out_shape=jax.ShapeDtypeStruct((N, D), jnp.float32),
    )(tg, center16, w1, w2, w3, wproj, res)


def _tc_seg_mean(h0, R, seg_blocks):
    d = h0.shape[1]

    def body(h_ref, o_ref):
        i = pl.program_id(0)

        @pl.when(i == 0)
        def _init():
            o_ref[...] = jnp.zeros_like(o_ref)

        s = jnp.sum(h_ref[...], axis=0, keepdims=True) * (1.0 / 25000.0)
        seg = i // seg_blocks
        rows = lax.broadcasted_iota(jnp.int32, (8, 1), 0)
        o_ref[...] += jnp.where(rows == seg, s, 0.0)

    return pl.pallas_call(
        body,
        grid=(2 * seg_blocks,),
        in_specs=[_row_bs(R, d)],
        out_specs=_full_bs((8, d)),
        out_shape=jax.ShapeDtypeStruct((8, d), jnp.float32),
    )(h0)


def _tc_dcd_head(mean8, fw1, fw2, f1w1, f1w2):
    d = mean8.shape[1]

    def body(m_ref, a1_ref, a2_ref, b1_ref, b2_ref, om_ref, ow_ref):
        m = m_ref[...]
        om_ref[...] = _gelu(_dot(_gelu(_dot(m, a1_ref[...])), a2_ref[...]))
        ow_ref[...] = jax.nn.sigmoid(_gelu(_dot(_gelu(_dot(m, b1_ref[...])), b2_ref[...])))

    return pl.pallas_call(
        body,
        grid=(1,),
        in_specs=[_full_bs(mean8.shape), _full_bs(fw1.shape), _full_bs(fw2.shape),
                  _full_bs(f1w1.shape), _full_bs(f1w2.shape)],
        out_specs=(_full_bs((8, d)), _full_bs((8, d))),
        out_shape=(jax.ShapeDtypeStruct((8, d), jnp.float32),
                   jax.ShapeDtypeStruct((8, d), jnp.float32)),
    )(mean8, fw1, fw2, f1w1, f1w2)


def _tc_final(h0, h1g, om, ow, cm, R, seg_blocks):
    d = h0.shape[1]
    N = N0

    def body(h0_ref, h1_ref, om_ref, ow_ref, cm_ref, o_ref):
        seg = pl.program_id(0) // seg_blocks
        rows = lax.broadcasted_iota(jnp.int32, (8, 1), 0)
        msk = rows == seg
        om_s = jnp.sum(jnp.where(msk, om_ref[...], 0.0), axis=0, keepdims=True)
        ow_s = jnp.sum(jnp.where(msk, ow_ref[...], 0.0), axis=0, keepdims=True)
        h0v = h0_ref[...]
        o_ref[...] = (ow_s * 0.5 + 0.75) * h0v * cm_ref[...] + om_s * cm_ref[...] + h1_ref[...]

    return pl.pallas_call(
        body,
        grid=(N // R,),
        in_specs=[_row_bs(R, d), _row_bs(R, d), _full_bs((8, d)), _full_bs((8, d)),
                  _full_bs((1, d))],
        out_specs=_row_bs(R, d),
        out_shape=jax.ShapeDtypeStruct((N, d), jnp.float32),
    )(h0, h1g, om, ow, cm)


def _bns(g):
    return g * _S


def _emb(w, shape, r0=0, c0=0):
    out = jnp.zeros(shape, jnp.float32)
    return out.at[r0:r0 + w.shape[0], c0:c0 + w.shape[1]].set(w)


def _padv(g, n=D):
    return jnp.concatenate([g, jnp.zeros((n - g.shape[0],), jnp.float32)])


def _embed3_w(p):
    return p["w1"] * _bns(p["g1"])[None, :], p["w2"] * _bns(p["g2"])[None, :], p["w3"]


def _block_apply(f, knn_km, p, R, Rg, gchunk):
    N = f.shape[0]
    m = p["mlp"]
    f, y = _tc_mlp_mm(f, _emb(m["w1"], (D, m["w1"].shape[1])), m["b1"],
                      _emb(m["w2"] * _bns(m["g"])[None, :], (m["w2"].shape[0], D)),
                      _emb(p["lfps"][0]["w"], (D, D)), R)
    for i in range(4):
        g3 = _gather_rows(y, knn_km, gchunk).reshape(KNN, N, D)
        mlp = None
        if i % 2 == 1:
            mm = p["mlps"][i // 2]
            mlp = (_emb(mm["w1"], (D, mm["w1"].shape[1])), mm["b1"],
                   _emb(mm["w2"] * _bns(mm["g"])[None, :], (mm["w2"].shape[0], D)))
        nxtw = _emb(p["lfps"][i + 1]["w"], (D, D)) if i < 3 else None
        svec = _padv(_bns(p["lfps"][i]["g"]))
        if nxtw is not None:
            f, y = _tc_lfp_step(g3, y, f, svec, mlp, nxtw, Rg)
        else:
            f = _tc_lfp_step(g3, y, f, svec, mlp, None, Rg)
    return f


def kernel(x, xyz, knn0, knn1, ids1, back_nn1, params):
    p0, p1 = params["s0"], params["s1"]
    f32 = jnp.float32

    knn0i = knn0.astype(jnp.int32)
    knn1i = knn1.astype(jnp.int32)
    knn0_km = jnp.pad(knn0i.T, ((0, 0), (0, N0P - N0))).reshape(-1)
    knn1_km = jnp.pad(knn1i.T, ((0, 0), (0, N1P - N1))).reshape(-1)
    ids1_pad = jnp.pad(ids1.astype(jnp.int32), (0, N1P - N1))
    back_pad = jnp.pad(back_nn1.astype(jnp.int32), (0, N0BP - N0))

    xp = jnp.pad(x, ((0, N0P - N0), (0, 0)))
    xyzp = jnp.pad(xyz, ((0, N0P - N0), (0, 0)))
    xup = jnp.concatenate([xyzp, xp, jnp.zeros((N0P, D - 7), f32)], axis=1)
    c0 = jnp.concatenate([xyzp, jnp.zeros((N0P, D - 3), f32)], axis=1)

    w1x, w2x, w3x = _embed3_w(p0["xemb"])
    t0 = _tc_embed3(xp, w1x, w2x, _emb(w3x, (w3x.shape[0], D), c0=16), xup, 1024)

    w1n, w2n, w3n = _embed3_w(p0["nbr"])
    c16 = jnp.concatenate([xyzp, jnp.zeros((N0P, 13), f32)], axis=1)
    tg0 = _gather_rows(t0, knn0_km, 400).reshape(KNN, N0P, D)
    f0 = _tc_edge0(tg0, c16, _emb(w1n, (16, w1n.shape[1])), w2n, w3n,
                   _bns(p0["nbr_bn_g"]), N0P, 400)
    f0 = _block_apply(f0, knn0_km, p0["blk"], 1024, 400, 400)

    a, y5 = _tc_mm2(f0, _emb(p1["skip_w"] * _bns(p1["skip_g"])[None, :], (D, D)),
                    _emb(p1["lfp_w"], (D, D)), 1024)
    knn0f = lax.bitcast_convert_type(
        jnp.pad(knn0i, ((0, N0P - N0), (0, D - KNN))), jnp.float32)
    y5s, a_s, xu1, knn5f = _gather_rows4((y5, a, c0, knn0f), ids1_pad)
    knn5 = lax.bitcast_convert_type(knn5f, jnp.int32)[:, :KNN]
    knn5_km = knn5.T.reshape(-1)
    g5 = _gather_rows(y5, knn5_km, 400).reshape(KNN, N1P, D)
    f1 = _tc_lfp_max(g5, y5s, a_s, _bns(p1["lfp_g"]), 512)

    w1x1, w2x1, w3x1 = _embed3_w(p1["xemb"])
    t1c = _tc_embed3(f1, w1x1, w2x1, _emb(w3x1, (w3x1.shape[0], D), c0=16), xu1, 512)

    w1n1, w2n1, w3n1 = _embed3_w(p1["nbr"])
    tg1 = _gather_rows(t1c, knn1_km, 400).reshape(KNN, N1P, D)
    f1 = _tc_edge1(tg1, xu1[:, :16], _emb(w1n1, (16, w1n1.shape[1])), w2n1, w3n1,
                   p1["nbr_proj_w"] * _bns(p1["nbr_bn_g"])[None, :], f1, N1P, 512)
    f1 = _block_apply(f1, knn1_km, p1["blk"], 512, 512, 400)

    t1 = _tc_matmul(f1, p1["post_w"] * _bns(p1["post_bn_g"])[:, None], 512)
    h1g = _gather_rows(t1, back_pad, 400)
    h0 = _tc_matmul(f0, _emb(p0["post_w"] * _bns(p0["post_bn_g"])[:, None], (D, D)), 1024)

    mean8 = _tc_seg_mean(h0, 200, 125)
    fc, fc1 = p0["dcd"]["fc"], p0["dcd"]["fc1"]
    om, ow = _tc_dcd_head(mean8, fc["w1"] * _bns(fc["g1"])[None, :], fc["w2"],
                          fc1["w1"] * _bns(fc1["g1"])[None, :], fc1["w2"])
    return _tc_final(h0, h1g, om, ow, params["channel_matric"], 1000, 25)


# speedup vs baseline: 1.0039x; 1.0039x over previous
import functools

import jax
import jax.numpy as jnp
from jax import lax
from jax.experimental import pallas as pl
from jax.experimental.pallas import tpu as pltpu
from jax.experimental.pallas import tpu_sc as plsc

N0, N1, KNN = 50000, 12500, 16
N0P = 51200
N1P = 12800
N0BP = 51200
_S = float((1.0 + 1e-5) ** -0.5)
_NW = 32
D = 128


def _gelu(v):
    return 0.5 * v * (1.0 + lax.erf(v * 0.7071067811865476))


def _sc_gather_fn(V, B, C, dt):
    b_per_w = B // _NW
    nch = b_per_w // C
    mesh = plsc.VectorSubcoreMesh(core_axis_name="c", subcore_axis_name="s")

    @functools.partial(
        pl.kernel,
        out_type=jax.ShapeDtypeStruct((B, D), dt),
        mesh=mesh,
        scratch_types=[
            pltpu.VMEM((C,), jnp.int32),
            pltpu.VMEM((C,), jnp.int32),
            pltpu.VMEM((C, D), dt),
            pltpu.VMEM((C, D), dt),
            pltpu.SemaphoreType.DMA,
            pltpu.SemaphoreType.DMA,
            pltpu.SemaphoreType.DMA,
            pltpu.SemaphoreType.DMA,
        ],
    )
    def gather_k(table_hbm, idx_hbm, out_hbm, i0, i1, r0, r1, g0, g1, w0, w1):
        wid = lax.axis_index("s") * 2 + lax.axis_index("c")
        wbase = wid * b_per_w
        ibufs, rbufs, gsems, wsems = (i0, i1), (r0, r1), (g0, g1), (w0, w1)

        def start_gather(b, c):
            base = wbase + c * C
            pltpu.sync_copy(idx_hbm.at[pl.ds(base, C)], ibufs[b])
            pltpu.async_copy(table_hbm.at[ibufs[b]], rbufs[b], gsems[b])

        if nch < 2:
            pltpu.sync_copy(idx_hbm.at[pl.ds(wbase, C)], i0)
            pltpu.async_copy(table_hbm.at[i0], r0, g0).wait()
            pltpu.sync_copy(r0, out_hbm.at[pl.ds(wbase, C)])
            return

        start_gather(0, 0)
        start_gather(1, 1)

        def pair_body(j, carry):
            for b in range(2):
                c = 2 * j + b
                base = wbase + c * C
                pltpu.make_async_copy(table_hbm.at[pl.ds(0, C)], rbufs[b],
                                      gsems[b]).wait()
                pltpu.async_copy(rbufs[b], out_hbm.at[pl.ds(base, C)], wsems[b])
                pltpu.make_async_copy(rbufs[b], out_hbm.at[pl.ds(base, C)],
                                      wsems[b]).wait()

                @pl.when(c + 2 < nch)
                def _next():
                    start_gather(b, c + 2)

            return carry

        lax.fori_loop(0, nch // 2, pair_body, 0)

    return gather_k


def _gather_rows(table, idx, chunk):
    V, d = table.shape
    B = idx.shape[0]
    assert d == D and B % (_NW * chunk) == 0 and chunk % 8 == 0, (V, d, B, chunk)
    return _sc_gather_fn(V, B, chunk, table.dtype)(table, idx)


def _sc_gather4_fn(V, B, C):
    b_per_w = B // _NW
    assert b_per_w == C
    mesh = plsc.VectorSubcoreMesh(core_axis_name="c", subcore_axis_name="s")
    out4 = tuple(jax.ShapeDtypeStruct((B, D), jnp.float32) for _ in range(4))

    @functools.partial(
        pl.kernel,
        out_type=out4,
        mesh=mesh,
        scratch_types=[
            pltpu.VMEM((C,), jnp.int32),
            pltpu.VMEM((C, D), jnp.float32),
            pltpu.SemaphoreType.DMA,
        ],
    )
    def gather4_k(t0_h, t1_h, t2_h, t3_h, idx_hbm, o0, o1, o2, o3,
                  idx_v, rows_v, sem):
        wid = lax.axis_index("s") * 2 + lax.axis_index("c")
        wbase = wid * b_per_w
        pltpu.sync_copy(idx_hbm.at[pl.ds(wbase, C)], idx_v)
        for t_h, o_h in ((t0_h, o0), (t1_h, o1), (t2_h, o2), (t3_h, o3)):
            pltpu.async_copy(t_h.at[idx_v], rows_v, sem).wait()
            pltpu.sync_copy(rows_v, o_h.at[pl.ds(wbase, C)])

    return gather4_k


def _gather_rows4(tables, idx):
    B = idx.shape[0]
    return _sc_gather4_fn(tables[0].shape[0], B, B // _NW)(*tables, idx)


def _row_bs(R, d):
    return pl.BlockSpec((R, d), lambda i: (i, 0))


def _full_bs(shape):
    nd = len(shape)
    return pl.BlockSpec(shape, lambda i: (0,) * nd)


def _edge_bs(R):
    return pl.BlockSpec((KNN, R, D), lambda i: (0, i, 0))


def _dot(a, b):
    return jnp.dot(a, b, preferred_element_type=jnp.float32)


def _tc_matmul(x, w, R):
    N, Di = x.shape
    Do = w.shape[1]

    def body(x_ref, w_ref, o_ref):
        o_ref[...] = _dot(x_ref[...], w_ref[...])

    return pl.pallas_call(
        body,
        grid=(N // R,),
        in_specs=[_row_bs(R, Di), _full_bs(w.shape)],
        out_specs=_row_bs(R, Do),
        out_shape=jax.ShapeDtypeStruct((N, Do), jnp.float32),
    )(x, w)


def _tc_mm2(x, wa, wb, R):
    N, Di = x.shape

    def body(x_ref, wa_ref, wb_ref, oa_ref, ob_ref):
        v = x_ref[...]
        oa_ref[...] = _dot(v, wa_ref[...])
        ob_ref[...] = _dot(v, wb_ref[...])

    return pl.pallas_call(
        body,
        grid=(N // R,),
        in_specs=[_row_bs(R, Di), _full_bs(wa.shape), _full_bs(wb.shape)],
        out_specs=(_row_bs(R, wa.shape[1]), _row_bs(R, wb.shape[1])),
        out_shape=(jax.ShapeDtypeStruct((N, wa.shape[1]), jnp.float32),
                   jax.ShapeDtypeStruct((N, wb.shape[1]), jnp.float32)),
    )(x, wa, wb)


def _tc_mlp_mm(x, w1, b1, w2s, nxtw, R):
    N, d = x.shape

    def body(x_ref, w1_ref, b1_ref, w2_ref, nw_ref, o_ref, y_ref):
        v = x_ref[...]
        t = v + _dot(_gelu(_dot(v, w1_ref[...]) + b1_ref[...]), w2_ref[...])
        o_ref[...] = t
        y_ref[...] = _dot(t, nw_ref[...])

    return pl.pallas_call(
        body,
        grid=(N // R,),
        in_specs=[_row_bs(R, d), _full_bs(w1.shape), _full_bs((1,) + b1.shape),
                  _full_bs(w2s.shape), _full_bs(nxtw.shape)],
        out_specs=(_row_bs(R, d), _row_bs(R, d)),
        out_shape=(jax.ShapeDtypeStruct((N, d), jnp.float32),
                   jax.ShapeDtypeStruct((N, d), jnp.float32)),
    )(x, w1, b1[None, :], w2s, nxtw)


def _tc_lfp_step(g3, y, f, svec, mlp, nxtw, R):
    N = y.shape[0]
    specs = [_edge_bs(R), _row_bs(R, D), _row_bs(R, D), _full_bs((1, D))]
    args = [g3, y, f, svec[None, :]]
    if mlp is not None:
        w1, b1, w2s = mlp
        specs += [_full_bs(w1.shape), _full_bs((1,) + b1.shape), _full_bs(w2s.shape)]
        args += [w1, b1[None, :], w2s]
    if nxtw is not None:
        specs += [_full_bs(nxtw.shape)]
        args += [nxtw]

    def body(*refs):
        g_ref, y_ref, f_ref, s_ref = refs[:4]
        rest = list(refs[4:-2] if nxtw is not None else refs[4:-1])
        acc = g_ref[0]
        for k in range(1, KNN):
            acc = jnp.maximum(acc, g_ref[k])
        t = f_ref[...] + (acc - y_ref[...]) * s_ref[...]
        if mlp is not None:
            w1_ref, b1_ref, w2_ref = rest[:3]
            t = t + _dot(_gelu(_dot(t, w1_ref[...]) + b1_ref[...]), w2_ref[...])
        if nxtw is not None:
            refs[-2][...] = t
            refs[-1][...] = _dot(t, refs[4 + (3 if mlp is not None else 0)][...])
        else:
            refs[-1][...] = t

    if nxtw is not None:
        out_specs = (_row_bs(R, D), _row_bs(R, D))
        out_shape = (jax.ShapeDtypeStruct((N, D), jnp.float32),
                     jax.ShapeDtypeStruct((N, D), jnp.float32))
    else:
        out_specs = _row_bs(R, D)
        out_shape = jax.ShapeDtypeStruct((N, D), jnp.float32)

    return pl.pallas_call(
        body,
        grid=(N // R,),
        in_specs=specs,
        out_specs=out_specs,
        out_shape=out_shape,
    )(*args)


def _tc_embed3(x, w1, w2, w3, add, R):
    N, Di = x.shape
    Do = w3.shape[1]

    def body(x_ref, w1_ref, w2_ref, w3_ref, a_ref, o_ref):
        h = _gelu(_dot(x_ref[...], w1_ref[...]))
        h = _gelu(_dot(h, w2_ref[...]))
        o_ref[...] = _dot(h, w3_ref[...]) + a_ref[...]

    return pl.pallas_call(
        body,
        grid=(N // R,),
        in_specs=[_row_bs(R, Di), _full_bs(w1.shape), _full_bs(w2.shape),
                  _full_bs(w3.shape), _row_bs(R, Do)],
        out_specs=_row_bs(R, Do),
        out_shape=jax.ShapeDtypeStruct((N, Do), jnp.float32),
    )(x, w1, w2, w3, add)


def _tc_lfp_max(g3, y, res, svec, R):
    N = y.shape[0]

    def body(g_ref, y_ref, r_ref, s_ref, o_ref):
        acc = g_ref[0]
        for k in range(1, KNN):
            acc = jnp.maximum(acc, g_ref[k])
        o_ref[...] = r_ref[...] + (acc.astype(jnp.float32) - y_ref[...]) * s_ref[...]

    return pl.pallas_call(
        body,
        grid=(N // R,),
        in_specs=[_edge_bs(R), _row_bs(R, D), _row_bs(R, D), _full_bs((1, D))],
        out_specs=_row_bs(R, D),
        out_shape=jax.ShapeDtypeStruct((N, D), jnp.float32),
    )(g3, y, res, svec[None, :])


def _tc_edge0(tg, center, w1, w2, w3, esel, svec, N, R):
    def body(tg_ref, c_ref, w1_ref, w2_ref, w3_ref, e_ref, s_ref, o_ref):
        c = c_ref[...]
        acc = None
        for k in range(KNN):
            e = tg_ref[k] - c
            h = _gelu(_dot(e, w1_ref[...]))
            h = _gelu(_dot(h, w2_ref[...]))
            h = _dot(h, w3_ref[...]) + _dot(e, e_ref[...])
            acc = h if acc is None else jnp.maximum(acc, h)
        o_ref[...] = acc * s_ref[...]

    return pl.pallas_call(
        body,
        grid=(N // R,),
        in_specs=[_edge_bs(R), _row_bs(R, D), _full_bs(w1.shape), _full_bs(w2.shape),
                  _full_bs(w3.shape), _full_bs(esel.shape), _full_bs((1, D))],
        out_specs=_row_bs(R, D),
        out_shape=jax.ShapeDtypeStruct((N, D), jnp.float32),
    )(tg, center, w1, w2, w3, esel, svec[None, :])


def _tc_edge1(tg, center, w1, w2, w3, esel, wproj, res, N, R):
    def body(tg_ref, c_ref, w1_ref, w2_ref, w3_ref, e_ref, wp_ref, r_ref, o_ref):
        c = c_ref[...]
        acc = None
        for k in range(KNN):
            e = tg_ref[k] - c
            h = _gelu(_dot(e, w1_ref[...]))
            h = _gelu(_dot(h, w2_ref[...]))
            h = _dot(h, w3_ref[...]) + _dot(e, e_ref[...])
            acc = h if acc is None else jnp.maximum(acc, h)
        o_ref[...] = r_ref[...] + _dot(acc, wp_ref[...])

    return pl.pallas_call(
        body,
        grid=(N // R,),
        in_specs=[_edge_bs(R), _row_bs(R, D), _full_bs(w1.shape), _full_bs(w2.shape),
                  _full_bs(w3.shape), _full_bs(esel.shape), _full_bs(wproj.shape),
                  _row_bs(R, D)],
        out_specs=_row_bs(R, D),
        out_shape=jax.ShapeDtypeStruct((N, D), jnp.float32),
    )(tg, center, w1, w2, w3, esel, wproj, res)


def _tc_seg_mean(h0, R, seg_blocks):
    d = h0.shape[1]

    def body(h_ref, o_ref):
        i = pl.program_id(0)

        @pl.when(i == 0)
        def _init():
            o_ref[...] = jnp.zeros_like(o_ref)

        s = jnp.sum(h_ref[...], axis=0, keepdims=True) * (1.0 / 25000.0)
        seg = i // seg_blocks
        rows = lax.broadcasted_iota(jnp.int32, (8, 1), 0)
        o_ref[...] += jnp.where(rows == seg, s, 0.0)

    return pl.pallas_call(
        body,
        grid=(2 * seg_blocks,),
        in_specs=[_row_bs(R, d)],
        out_specs=_full_bs((8, d)),
        out_shape=jax.ShapeDtypeStruct((8, d), jnp.float32),
    )(h0)


def _tc_dcd_head(mean8, fw1, fw2, f1w1, f1w2):
    d = mean8.shape[1]

    def body(m_ref, a1_ref, a2_ref, b1_ref, b2_ref, om_ref, ow_ref):
        m = m_ref[...]
        om_ref[...] = _gelu(_dot(_gelu(_dot(m, a1_ref[...])), a2_ref[...]))
        ow_ref[...] = jax.nn.sigmoid(_gelu(_dot(_gelu(_dot(m, b1_ref[...])), b2_ref[...])))

    return pl.pallas_call(
        body,
        grid=(1,),
        in_specs=[_full_bs(mean8.shape), _full_bs(fw1.shape), _full_bs(fw2.shape),
                  _full_bs(f1w1.shape), _full_bs(f1w2.shape)],
        out_specs=(_full_bs((8, d)), _full_bs((8, d))),
        out_shape=(jax.ShapeDtypeStruct((8, d), jnp.float32),
                   jax.ShapeDtypeStruct((8, d), jnp.float32)),
    )(mean8, fw1, fw2, f1w1, f1w2)


def _tc_final(h0, h1g, om, ow, cm, R, seg_blocks):
    d = h0.shape[1]
    N = N0

    def body(h0_ref, h1_ref, om_ref, ow_ref, cm_ref, o_ref):
        seg = pl.program_id(0) // seg_blocks
        rows = lax.broadcasted_iota(jnp.int32, (8, 1), 0)
        msk = rows == seg
        om_s = jnp.sum(jnp.where(msk, om_ref[...], 0.0), axis=0, keepdims=True)
        ow_s = jnp.sum(jnp.where(msk, ow_ref[...], 0.0), axis=0, keepdims=True)
        h0v = h0_ref[...]
        o_ref[...] = (ow_s * 0.5 + 0.75) * h0v * cm_ref[...] + om_s * cm_ref[...] + h1_ref[...]

    return pl.pallas_call(
        body,
        grid=(N // R,),
        in_specs=[_row_bs(R, d), _row_bs(R, d), _full_bs((8, d)), _full_bs((8, d)),
                  _full_bs((1, d))],
        out_specs=_row_bs(R, d),
        out_shape=jax.ShapeDtypeStruct((N, d), jnp.float32),
    )(h0, h1g, om, ow, cm)


def _bns(g):
    return g * _S


def _emb(w, shape, r0=0, c0=0):
    out = jnp.zeros(shape, jnp.float32)
    return out.at[r0:r0 + w.shape[0], c0:c0 + w.shape[1]].set(w)


def _padv(g, n=D):
    return jnp.concatenate([g, jnp.zeros((n - g.shape[0],), jnp.float32)])


def _embed3_w(p):
    return p["w1"] * _bns(p["g1"])[None, :], p["w2"] * _bns(p["g2"])[None, :], p["w3"]


def _block_apply(f, knn_km, p, R, Rg, gchunk):
    N = f.shape[0]
    m = p["mlp"]
    f, y = _tc_mlp_mm(f, _emb(m["w1"], (D, m["w1"].shape[1])), m["b1"],
                      _emb(m["w2"] * _bns(m["g"])[None, :], (m["w2"].shape[0], D)),
                      _emb(p["lfps"][0]["w"], (D, D)), R)
    for i in range(4):
        g3 = _gather_rows(y, knn_km, gchunk).reshape(KNN, N, D)
        mlp = None
        if i % 2 == 1:
            mm = p["mlps"][i // 2]
            mlp = (_emb(mm["w1"], (D, mm["w1"].shape[1])), mm["b1"],
                   _emb(mm["w2"] * _bns(mm["g"])[None, :], (mm["w2"].shape[0], D)))
        nxtw = _emb(p["lfps"][i + 1]["w"], (D, D)) if i < 3 else None
        svec = _padv(_bns(p["lfps"][i]["g"]))
        if nxtw is not None:
            f, y = _tc_lfp_step(g3, y, f, svec, mlp, nxtw, Rg)
        else:
            f = _tc_lfp_step(g3, y, f, svec, mlp, None, Rg)
    return f


def kernel(x, xyz, knn0, knn1, ids1, back_nn1, params):
    p0, p1 = params["s0"], params["s1"]
    f32 = jnp.float32

    knn0i = knn0.astype(jnp.int32)
    knn1i = knn1.astype(jnp.int32)
    knn0_km = jnp.pad(knn0i.T, ((0, 0), (0, N0P - N0))).reshape(-1)
    knn1_km = jnp.pad(knn1i.T, ((0, 0), (0, N1P - N1))).reshape(-1)
    ids1_pad = jnp.pad(ids1.astype(jnp.int32), (0, N1P - N1))
    back_pad = jnp.pad(back_nn1.astype(jnp.int32), (0, N0BP - N0))

    xp = jnp.pad(x, ((0, N0P - N0), (0, 0)))
    xyzp = jnp.pad(xyz, ((0, N0P - N0), (0, 0)))
    xup = jnp.concatenate([xyzp, xp, jnp.zeros((N0P, D - 7), f32)], axis=1)
    c0 = jnp.concatenate([xyzp, jnp.zeros((N0P, D - 3), f32)], axis=1)

    w1x, w2x, w3x = _embed3_w(p0["xemb"])
    t0 = _tc_embed3(xp, w1x, w2x, _emb(w3x, (w3x.shape[0], D), c0=16), xup, 1024)

    w1n, w2n, w3n = _embed3_w(p0["nbr"])
    esel0 = _emb(jnp.eye(64, dtype=f32), (D, D), r0=16)
    tg0 = _gather_rows(t0, knn0_km, 400).reshape(KNN, N0P, D)
    f0 = _tc_edge0(tg0, c0, _emb(w1n, (D, w1n.shape[1])), w2n,
                   _emb(w3n, (w3n.shape[0], D)), esel0, _padv(_bns(p0["nbr_bn_g"])),
                   N0P, 400)
    f0 = _block_apply(f0, knn0_km, p0["blk"], 1024, 400, 400)

    a, y5 = _tc_mm2(f0, _emb(p1["skip_w"] * _bns(p1["skip_g"])[None, :], (D, D)),
                    _emb(p1["lfp_w"], (D, D)), 1024)
    knn0f = lax.bitcast_convert_type(
        jnp.pad(knn0i, ((0, N0P - N0), (0, D - KNN))), jnp.float32)
    y5s, a_s, xu1, knn5f = _gather_rows4((y5, a, c0, knn0f), ids1_pad)
    knn5 = lax.bitcast_convert_type(knn5f, jnp.int32)[:, :KNN]
    knn5_km = knn5.T.reshape(-1)
    g5 = _gather_rows(y5, knn5_km, 400).reshape(KNN, N1P, D)
    f1 = _tc_lfp_max(g5, y5s, a_s, _bns(p1["lfp_g"]), 512)

    w1x1, w2x1, w3x1 = _embed3_w(p1["xemb"])
    t1c = _tc_embed3(f1, w1x1, w2x1, _emb(w3x1, (w3x1.shape[0], D), c0=16), xu1, 512)

    w1n1, w2n1, w3n1 = _embed3_w(p1["nbr"])
    esel1 = _emb(jnp.eye(32, dtype=f32), (D, 32), r0=16)
    tg1 = _gather_rows(t1c, knn1_km, 400).reshape(KNN, N1P, D)
    f1 = _tc_edge1(tg1, xu1, _emb(w1n1, (D, w1n1.shape[1])), w2n1, w3n1, esel1,
                   p1["nbr_proj_w"] * _bns(p1["nbr_bn_g"])[None, :], f1, N1P, 512)
    f1 = _block_apply(f1, knn1_km, p1["blk"], 512, 512, 400)

    t1 = _tc_matmul(f1, p1["post_w"] * _bns(p1["post_bn_g"])[:, None], 512)
    h1g = _gather_rows(t1, back_pad, 400)
    h0 = _tc_matmul(f0, _emb(p0["post_w"] * _bns(p0["post_bn_g"])[:, None], (D, D)), 1024)

    mean8 = _tc_seg_mean(h0, 200, 125)
    fc, fc1 = p0["dcd"]["fc"], p0["dcd"]["fc1"]
    om, ow = _tc_dcd_head(mean8, fc["w1"] * _bns(fc["g1"])[None, :], fc["w2"],
                          fc1["w1"] * _bns(fc1["g1"])[None, :], fc1["w2"])
    return _tc_final(h0, h1g, om, ow, params["channel_matric"], 1000, 25)
